# baseline (speedup 1.0000x reference)
import functools

import jax
import jax.numpy as jnp
from jax import lax
from jax.experimental import pallas as pl
from jax.experimental.pallas import tpu as pltpu
from jax.experimental.pallas import tpu_sc as plsc

B, S, D, N = 8, 512, 768, 32
NSPANS = B * N
NW = 32
SPW = NSPANS // NW
CH = 16
NBLK = S // CH
BPC = B // 2
SEG = S // 4
DV = 16
NVD = D // DV
DB = 128
NDB = D // DB
VPB = DB // DV
NEG = -3.0e38


def _sc_pool(x1d, starts, ends, rcp):
    mesh = plsc.VectorSubcoreMesh(core_axis_name="c", subcore_axis_name="s")

    @functools.partial(
        pl.kernel,
        mesh=mesh,
        out_type=(
            jax.ShapeDtypeStruct((NSPANS * 2 * D,), jnp.float32),
            jax.ShapeDtypeStruct((B * NBLK * D,), jnp.float32),
            jax.ShapeDtypeStruct((B * NBLK * D,), jnp.float32),
        ),
        scratch_types=[
            pltpu.VMEM((3 * CH * D,), jnp.float32),
            pltpu.VMEM((2 * D,), jnp.float32),
            pltpu.VMEM((24,), jnp.int32),
            pltpu.VMEM((24,), jnp.int32),
            pltpu.VMEM((528,), jnp.float32),
            pltpu.VMEM((2 * (SEG // CH) * D,), jnp.float32),
            pltpu.VMEM((SPW * 2 * D,), jnp.float32),
            pltpu.SemaphoreType.DMA,
            pltpu.SemaphoreType.DMA,
            pltpu.SemaphoreType.DMA,
        ],
    )
    def kern(x_hbm, st_hbm, en_hbm, rcp_hbm, out_hbm, bm_hbm, bs_hbm,
             buf, acc, stv, env, rcpv, blkout, outstage, sem0, sem1, sem2):
        cid = lax.axis_index("c")
        sid = lax.axis_index("s")

        def issue_from(ref, off, c):
            src = ref.at[pl.ds(off, CH * D)]

            @pl.when(c % 3 == 0)
            def _():
                pltpu.async_copy(src, buf.at[pl.ds(0, CH * D)], sem0)

            @pl.when(c % 3 == 1)
            def _():
                pltpu.async_copy(src, buf.at[pl.ds(CH * D, CH * D)], sem1)

            @pl.when(c % 3 == 2)
            def _():
                pltpu.async_copy(src, buf.at[pl.ds(2 * CH * D, CH * D)], sem2)

        def wait(c):
            dummy = x_hbm.at[pl.ds(0, CH * D)]

            @pl.when(c % 3 == 0)
            def _():
                pltpu.make_async_copy(
                    dummy, buf.at[pl.ds(0, CH * D)], sem0).wait()

            @pl.when(c % 3 == 1)
            def _():
                pltpu.make_async_copy(
                    dummy, buf.at[pl.ds(CH * D, CH * D)], sem1).wait()

            @pl.when(c % 3 == 2)
            def _():
                pltpu.make_async_copy(
                    dummy, buf.at[pl.ds(2 * CH * D, CH * D)], sem2).wait()

        p1_batch = BPC * cid + sid // 4
        p1_q = sid % 4
        seg0 = p1_batch * S + p1_q * SEG
        g0 = p1_batch * NBLK + p1_q * (SEG // CH)

        NB = SEG // CH
        issue_from(x_hbm, (seg0 + 0 * CH) * D, 0)
        issue_from(x_hbm, (seg0 + 1 * CH) * D, 1)

        def blk_body(j, _):
            @pl.when(j + 2 < NB)
            def _():
                issue_from(x_hbm, (seg0 + (j + 2) * CH) * D, j + 2)

            wait(j)
            boff = (j % 3) * (CH * D)

            def db_body(db, _):
                col = boff + db * DB
                bmax = [buf[pl.ds(col + i * DV, DV)] for i in range(VPB)]
                bsum = list(bmax)
                for r in range(1, CH):
                    for i in range(VPB):
                        xv = buf[pl.ds(col + r * D + i * DV, DV)]
                        bmax[i] = jnp.maximum(bmax[i], xv)
                        bsum[i] = bsum[i] + xv
                ocol = j * D + db * DB
                for i in range(VPB):
                    blkout[pl.ds(ocol + i * DV, DV)] = bmax[i]
                    blkout[pl.ds(NB * D + ocol + i * DV, DV)] = bsum[i]
                return 0

            lax.fori_loop(0, NDB, db_body, 0)
            return 0

        lax.fori_loop(0, NB, blk_body, 0)
        pltpu.sync_copy(blkout.at[pl.ds(0, NB * D)],
                        bm_hbm.at[pl.ds(g0 * D, NB * D)])
        pltpu.sync_copy(blkout.at[pl.ds(NB * D, NB * D)],
                        bs_hbm.at[pl.ds(g0 * D, NB * D)])
        plsc.subcore_barrier()

        base = cid * (NSPANS // 2) + sid * SPW
        pltpu.sync_copy(st_hbm.at[pl.ds(base, SPW)], stv.at[pl.ds(0, SPW)])
        pltpu.sync_copy(en_hbm.at[pl.ds(base, SPW)], env.at[pl.ds(0, SPW)])
        pltpu.sync_copy(rcp_hbm, rcpv)

        def span_body(k, _):
            start = stv[pl.ds(k, 16)][0]
            end = env[pl.ds(k, 16)][0]
            s_id = base + k
            batch = s_id // N
            n_rows = end - start + 1
            fb = (start + CH - 1) // CH
            lbp1 = (end + 1) // CH
            ni = jnp.maximum(lbp1 - fb, 0)
            nI = (ni + CH - 1) // CH
            total = 2 + 2 * nI
            ll = jnp.minimum(start, S - CH)
            lhi = jnp.minimum(fb * CH - 1, end)
            rlo = jnp.maximum(lbp1, fb) * CH
            rl = jnp.minimum(rlo, S - CH)
            xbase = batch * S
            gbase = batch * NBLK

            def job_params(c):
                is_l = c == 0
                is_r = c == 1
                t = c - 2
                is_m = jnp.logical_and(c >= 2, t < nI)
                is_e = jnp.logical_or(is_l, is_r)
                tt = jnp.where(is_m, t, t - nI)
                bo = jnp.minimum(fb + tt * CH, NBLK - CH)
                return is_l, is_r, is_e, is_m, tt, bo

            def issue2(c):
                is_l, is_r, is_e, is_m, tt, bo = job_params(c)
                eoff = (xbase + jnp.where(is_l, ll, rl)) * D
                goff = (gbase + bo) * D

                @pl.when(is_e)
                def _():
                    issue_from(x_hbm, eoff, c)

                @pl.when(jnp.logical_and(jnp.logical_not(is_e), is_m))
                def _():
                    issue_from(bm_hbm, goff, c)

                @pl.when(jnp.logical_and(jnp.logical_not(is_e),
                                         jnp.logical_not(is_m)))
                def _():
                    issue_from(bs_hbm, goff, c)

            issue2(0)
            issue2(1)

            def init_body(j, _):
                val = jnp.where(j < NVD, NEG, jnp.float32(0.0))
                acc[pl.ds(j * DV, DV)] = jnp.full((DV,), jnp.float32(0.0)) + val
                return 0

            lax.fori_loop(0, 2 * NVD, init_body, 0)

            def job_body(c, _):
                @pl.when(c + 2 < total)
                def _():
                    issue2(c + 2)

                wait(c)
                is_l, is_r, is_e, is_m, tt, bo = job_params(c)
                boff = (c % 3) * (CH * D)
                vlo = jnp.where(is_l, start,
                                jnp.where(is_r, rlo, fb + tt * CH))
                vhi = jnp.where(is_l, lhi, jnp.where(is_r, end, lbp1 - 1))
                abase = jnp.where(is_l, ll, jnp.where(is_r, rl, bo))
                rlo_r = vlo - abase
                rhi_r = jnp.minimum(vhi - abase, CH - 1)

                @pl.when(is_e)
                def _edge():
                    def db_edge(db, _):
                        col = db * DB
                        init = tuple(
                            acc[pl.ds(col + i * DV, DV)] for i in range(VPB)
                        ) + tuple(
                            acc[pl.ds(D + col + i * DV, DV)]
                            for i in range(VPB))

                        def row_body(r, carry):
                            off = boff + r * D + col
                            out = []
                            for i in range(VPB):
                                xv = buf[pl.ds(off + i * DV, DV)]
                                out.append(jnp.maximum(carry[i], xv))
                            for i in range(VPB):
                                xv = buf[pl.ds(off + i * DV, DV)]
                                out.append(carry[VPB + i] + xv)
                            return tuple(out)

                        res = lax.fori_loop(rlo_r, rhi_r + 1, row_body, init)
                        for i in range(VPB):
                            acc[pl.ds(col + i * DV, DV)] = res[i]
                            acc[pl.ds(D + col + i * DV, DV)] = res[VPB + i]
                        return 0

                    lax.fori_loop(0, NDB, db_edge, 0)

                @pl.when(jnp.logical_and(jnp.logical_not(is_e), is_m))
                def _imax():
                    def db_max(db, _):
                        col = db * DB
                        init = tuple(
                            acc[pl.ds(col + i * DV, DV)] for i in range(VPB))

                        def row_body(r, carry):
                            off = boff + r * D + col
                            return tuple(
                                jnp.maximum(carry[i],
                                            buf[pl.ds(off + i * DV, DV)])
                                for i in range(VPB))

                        res = lax.fori_loop(rlo_r, rhi_r + 1, row_body, init)
                        for i in range(VPB):
                            acc[pl.ds(col + i * DV, DV)] = res[i]
                        return 0

                    lax.fori_loop(0, NDB, db_max, 0)

                @pl.when(jnp.logical_and(jnp.logical_not(is_e),
                                         jnp.logical_not(is_m)))
                def _isum():
                    def db_sum(db, _):
                        col = db * DB
                        init = tuple(
                            acc[pl.ds(D + col + i * DV, DV)]
                            for i in range(VPB))

                        def row_body(r, carry):
                            off = boff + r * D + col
                            return tuple(
                                carry[i] + buf[pl.ds(off + i * DV, DV)]
                                for i in range(VPB))

                        res = lax.fori_loop(rlo_r, rhi_r + 1, row_body, init)
                        for i in range(VPB):
                            acc[pl.ds(D + col + i * DV, DV)] = res[i]
                        return 0

                    lax.fori_loop(0, NDB, db_sum, 0)

                return 0

            lax.fori_loop(0, total, job_body, 0)

            scale = rcpv[pl.ds(n_rows - 1, 16)][0]
            orow = k * 2 * D

            def fin_body(j, _):
                outstage[pl.ds(orow + j * DV, DV)] = acc[pl.ds(j * DV, DV)]
                outstage[pl.ds(orow + D + j * DV, DV)] = (
                    acc[pl.ds(D + j * DV, DV)] * scale)
                return 0

            lax.fori_loop(0, NVD, fin_body, 0)
            return 0

        lax.fori_loop(0, SPW, span_body, 0)
        pltpu.sync_copy(outstage,
                        out_hbm.at[pl.ds(base * 2 * D, SPW * 2 * D)])

    return kern(x1d, starts, ends, rcp)[0]


def _tc_proj(cat, W, b2):
    def body(c_ref, w_ref, b_ref, o_ref):
        o_ref[...] = lax.dot_general(
            c_ref[...], w_ref[...],
            dimension_numbers=(((1,), (1,)), ((), ())),
            preferred_element_type=jnp.float32,
        ) + b_ref[...]

    return pl.pallas_call(
        body,
        out_shape=jax.ShapeDtypeStruct((NSPANS, D), jnp.float32),
    )(cat, W, b2)


def kernel(sentence_repr, entity_span_indices, W, b):
    x1d = sentence_repr.reshape(B * S * D)
    esi = entity_span_indices.astype(jnp.int32).reshape(NSPANS, 2)
    rcp = 1.0 / jnp.arange(1, 529, dtype=jnp.float32)
    cat = _sc_pool(x1d, esi[:, 0], esi[:, 1], rcp).reshape(NSPANS, 2 * D)
    out = _tc_proj(cat, W, b.reshape(1, D))
    return out.reshape(B, N, D)


# speedup vs baseline: 1.1066x; 1.1066x over previous
import functools

import jax
import jax.numpy as jnp
from jax import lax
from jax.experimental import pallas as pl
from jax.experimental.pallas import tpu as pltpu
from jax.experimental.pallas import tpu_sc as plsc

B, S, D, N = 8, 512, 768, 32
NSPANS = B * N
NW = 32
SPW = NSPANS // NW
CH = 16
NBLK = S // CH
BPC = B // 2
SEG = S // 4
DV = 16
NVD = D // DV
DB = 128
NDB = D // DB
VPB = DB // DV
NEG = -3.0e38


def _sc_maxpool(x1d, starts, ends):
    mesh = plsc.VectorSubcoreMesh(core_axis_name="c", subcore_axis_name="s")

    @functools.partial(
        pl.kernel,
        mesh=mesh,
        out_type=(
            jax.ShapeDtypeStruct((NSPANS * D,), jnp.float32),
            jax.ShapeDtypeStruct((B * NBLK * D,), jnp.float32),
        ),
        scratch_types=[
            pltpu.VMEM((3 * CH * D,), jnp.float32),
            pltpu.VMEM((D,), jnp.float32),
            pltpu.VMEM((24,), jnp.int32),
            pltpu.VMEM((24,), jnp.int32),
            pltpu.VMEM(((SEG // CH) * D,), jnp.float32),
            pltpu.VMEM((SPW * D,), jnp.float32),
            pltpu.SemaphoreType.DMA,
            pltpu.SemaphoreType.DMA,
            pltpu.SemaphoreType.DMA,
        ],
    )
    def kern(x_hbm, st_hbm, en_hbm, out_hbm, bm_hbm,
             buf, acc, stv, env, blkout, outstage, sem0, sem1, sem2):
        cid = lax.axis_index("c")
        sid = lax.axis_index("s")

        def issue_from(ref, off, c):
            src = ref.at[pl.ds(off, CH * D)]

            @pl.when(c % 3 == 0)
            def _():
                pltpu.async_copy(src, buf.at[pl.ds(0, CH * D)], sem0)

            @pl.when(c % 3 == 1)
            def _():
                pltpu.async_copy(src, buf.at[pl.ds(CH * D, CH * D)], sem1)

            @pl.when(c % 3 == 2)
            def _():
                pltpu.async_copy(src, buf.at[pl.ds(2 * CH * D, CH * D)], sem2)

        def wait(c):
            dummy = x_hbm.at[pl.ds(0, CH * D)]

            @pl.when(c % 3 == 0)
            def _():
                pltpu.make_async_copy(
                    dummy, buf.at[pl.ds(0, CH * D)], sem0).wait()

            @pl.when(c % 3 == 1)
            def _():
                pltpu.make_async_copy(
                    dummy, buf.at[pl.ds(CH * D, CH * D)], sem1).wait()

            @pl.when(c % 3 == 2)
            def _():
                pltpu.make_async_copy(
                    dummy, buf.at[pl.ds(2 * CH * D, CH * D)], sem2).wait()

        p1_batch = BPC * cid + sid // 4
        p1_q = sid % 4
        seg0 = p1_batch * S + p1_q * SEG
        g0 = p1_batch * NBLK + p1_q * (SEG // CH)

        NB = SEG // CH
        issue_from(x_hbm, (seg0 + 0 * CH) * D, 0)
        issue_from(x_hbm, (seg0 + 1 * CH) * D, 1)

        def blk_body(j, _):
            @pl.when(j + 2 < NB)
            def _():
                issue_from(x_hbm, (seg0 + (j + 2) * CH) * D, j + 2)

            wait(j)
            boff = (j % 3) * (CH * D)

            def db_body(db, _):
                col = boff + db * DB
                bmax = [buf[pl.ds(col + i * DV, DV)] for i in range(VPB)]
                for r in range(1, CH):
                    for i in range(VPB):
                        xv = buf[pl.ds(col + r * D + i * DV, DV)]
                        bmax[i] = jnp.maximum(bmax[i], xv)
                ocol = j * D + db * DB
                for i in range(VPB):
                    blkout[pl.ds(ocol + i * DV, DV)] = bmax[i]
                return 0

            lax.fori_loop(0, NDB, db_body, 0)
            return 0

        lax.fori_loop(0, NB, blk_body, 0)
        pltpu.sync_copy(blkout, bm_hbm.at[pl.ds(g0 * D, NB * D)])
        plsc.subcore_barrier()

        base = cid * (NSPANS // 2) + sid * SPW
        pltpu.sync_copy(st_hbm.at[pl.ds(base, SPW)], stv.at[pl.ds(0, SPW)])
        pltpu.sync_copy(en_hbm.at[pl.ds(base, SPW)], env.at[pl.ds(0, SPW)])

        def span_body(k, _):
            start = stv[pl.ds(k, 16)][0]
            end = env[pl.ds(k, 16)][0]
            s_id = base + k
            batch = s_id // N
            fb = (start + CH - 1) // CH
            lbp1 = (end + 1) // CH
            ni = jnp.maximum(lbp1 - fb, 0)
            nI = (ni + CH - 1) // CH
            total = 2 + nI
            ll = jnp.minimum(start, S - CH)
            lhi = jnp.minimum(fb * CH - 1, end)
            rlo = jnp.maximum(lbp1, fb) * CH
            rl = jnp.minimum(rlo, S - CH)
            xbase = batch * S
            gbase = batch * NBLK

            def job_params(c):
                is_l = c == 0
                is_r = c == 1
                is_e = jnp.logical_or(is_l, is_r)
                tt = c - 2
                bo = jnp.minimum(fb + tt * CH, NBLK - CH)
                return is_l, is_r, is_e, tt, bo

            def issue2(c):
                is_l, is_r, is_e, tt, bo = job_params(c)
                eoff = (xbase + jnp.where(is_l, ll, rl)) * D
                goff = (gbase + bo) * D

                @pl.when(is_e)
                def _():
                    issue_from(x_hbm, eoff, c)

                @pl.when(jnp.logical_not(is_e))
                def _():
                    issue_from(bm_hbm, goff, c)

            issue2(0)
            issue2(1)

            def init_body(j, _):
                acc[pl.ds(j * DV, DV)] = jnp.full((DV,), NEG, jnp.float32)
                return 0

            lax.fori_loop(0, NVD, init_body, 0)

            def job_body(c, _):
                @pl.when(c + 2 < total)
                def _():
                    issue2(c + 2)

                wait(c)
                is_l, is_r, is_e, tt, bo = job_params(c)
                boff = (c % 3) * (CH * D)
                vlo = jnp.where(is_l, start,
                                jnp.where(is_r, rlo, fb + tt * CH))
                vhi = jnp.where(is_l, lhi, jnp.where(is_r, end, lbp1 - 1))
                abase = jnp.where(is_l, ll, jnp.where(is_r, rl, bo))
                rlo_r = vlo - abase
                rhi_r = jnp.minimum(vhi - abase, CH - 1)

                def db_max(db, _):
                    col = db * DB
                    init = tuple(
                        acc[pl.ds(col + i * DV, DV)] for i in range(VPB))

                    def row_body(r, carry):
                        off = boff + r * D + col
                        return tuple(
                            jnp.maximum(carry[i],
                                        buf[pl.ds(off + i * DV, DV)])
                            for i in range(VPB))

                    res = lax.fori_loop(rlo_r, rhi_r + 1, row_body, init)
                    for i in range(VPB):
                        acc[pl.ds(col + i * DV, DV)] = res[i]
                    return 0

                lax.fori_loop(0, NDB, db_max, 0)
                return 0

            lax.fori_loop(0, total, job_body, 0)

            orow = k * D

            def fin_body(j, _):
                outstage[pl.ds(orow + j * DV, DV)] = acc[pl.ds(j * DV, DV)]
                return 0

            lax.fori_loop(0, NVD, fin_body, 0)
            return 0

        lax.fori_loop(0, SPW, span_body, 0)
        pltpu.sync_copy(outstage, out_hbm.at[pl.ds(base * D, SPW * D)])

    return kern(x1d, starts, ends)[0]


def _tc_meanproj(x, starts_bn, ends_bn, W, b2):
    def body(st_ref, en_ref, x_ref, w_ref, b_ref, o_ref):
        st = st_ref[0, 0]
        en = en_ref[0, 0]
        pos = lax.broadcasted_iota(jnp.int32, (N, S), 1)
        mask = jnp.logical_and(pos >= st[:, None], pos <= en[:, None])
        maskf = mask.astype(jnp.float32)
        sums = jax.lax.dot_general(
            maskf, x_ref[0],
            dimension_numbers=(((1,), (0,)), ((), ())),
            preferred_element_type=jnp.float32)
        widths = (en - st + 1).astype(jnp.float32)
        mean = sums / widths[:, None]
        o_ref[0] = jax.lax.dot_general(
            mean, w_ref[...],
            dimension_numbers=(((1,), (1,)), ((), ())),
            preferred_element_type=jnp.float32) + b_ref[...]

    return pl.pallas_call(
        body,
        grid=(B,),
        in_specs=[
            pl.BlockSpec((1, 1, N), lambda i: (i, 0, 0)),
            pl.BlockSpec((1, 1, N), lambda i: (i, 0, 0)),
            pl.BlockSpec((1, S, D), lambda i: (i, 0, 0)),
            pl.BlockSpec((D, D), lambda i: (0, 1)),
            pl.BlockSpec((1, D), lambda i: (0, 0)),
        ],
        out_specs=pl.BlockSpec((1, N, D), lambda i: (i, 0, 0)),
        out_shape=jax.ShapeDtypeStruct((B, N, D), jnp.float32),
    )(starts_bn.reshape(B, 1, N), ends_bn.reshape(B, 1, N), x, W, b2)


def _tc_maxproj(mx, W, partial):
    def body(m_ref, w_ref, p_ref, o_ref):
        o_ref[...] = jax.lax.dot_general(
            m_ref[...], w_ref[...],
            dimension_numbers=(((1,), (1,)), ((), ())),
            preferred_element_type=jnp.float32) + p_ref[...]

    return pl.pallas_call(
        body,
        grid=(1,),
        in_specs=[
            pl.BlockSpec((NSPANS, D), lambda i: (0, 0)),
            pl.BlockSpec((D, D), lambda i: (0, 0)),
            pl.BlockSpec((NSPANS, D), lambda i: (0, 0)),
        ],
        out_specs=pl.BlockSpec((NSPANS, D), lambda i: (0, 0)),
        out_shape=jax.ShapeDtypeStruct((NSPANS, D), jnp.float32),
    )(mx, W, partial)


def kernel(sentence_repr, entity_span_indices, W, b):
    x1d = sentence_repr.reshape(B * S * D)
    esi = entity_span_indices.astype(jnp.int32)
    starts_bn = esi[..., 0]
    ends_bn = esi[..., 1]
    mx = _sc_maxpool(x1d, starts_bn.reshape(NSPANS), ends_bn.reshape(NSPANS))
    partial = _tc_meanproj(sentence_repr, starts_bn, ends_bn, W,
                           b.reshape(1, D))
    out = _tc_maxproj(mx.reshape(NSPANS, D), W,
                      partial.reshape(NSPANS, D))
    return out.reshape(B, N, D)
